# baseline (speedup 1.0000x reference)
import functools

import jax
import jax.numpy as jnp
from jax import lax
from jax.experimental import pallas as pl
from jax.experimental.pallas import tpu as pltpu
from jax.experimental.pallas import tpu_sc as plsc

NC = 2
NS = 16
NW = NC * NS
BLK = 128
DEG_L = 16
BM = 1000
VL = 16


def _sc_mesh():
    return plsc.VectorSubcoreMesh(core_axis_name="c", subcore_axis_name="s")


_SC_PARAMS = pltpu.CompilerParams(use_tc_tiling_on_sc=False,
                                  needs_layout_passes=False)


def _fill(ref, rows, cols, value):
    vec = jnp.full((VL,), value, jnp.float32)

    @pl.loop(0, rows)
    def _(i):
        for l in range(cols // VL):
            ref[i, pl.ds(l * VL, VL)] = vec


def _zero_stripe(zbuf, acc_sh, base, rows, sem):
    @pl.loop(0, rows, step=CH)
    def _(i):
        pltpu.async_copy(zbuf, acc_sh.at[pl.ds(base + i, CH)], sem)

    @pl.loop(0, rows, step=CH)
    def _(i):
        pltpu.make_async_copy(zbuf, acc_sh.at[pl.ds(base + i, CH)], sem).wait()


CH = 64


def _rsqrt16(d):
    half = jnp.full((VL,), 1.5, jnp.float32)
    hh = d * jnp.full((VL,), -0.5, jnp.float32)
    i = plsc.bitcast(d, jnp.int32)
    i = jnp.full((VL,), 0x5F3759DF, jnp.int32) - lax.shift_right_logical(
        i, jnp.full((VL,), 1, jnp.int32))
    y = plsc.bitcast(i, jnp.float32)
    for _ in range(3):
        y = y * (half + hh * y * y)
    return y


def _deg_kernel(n_pad, r):
    rpt = n_pad // NS

    @functools.partial(
        pl.kernel,
        out_type=jax.ShapeDtypeStruct((NC, n_pad, DEG_L), jnp.float32),
        mesh=_sc_mesh(),
        scratch_types=[
            pltpu.VMEM((r, BLK), jnp.int32),
            pltpu.VMEM((BLK, DEG_L), jnp.float32),
            pltpu.VMEM((CH, DEG_L), jnp.float32),
            pltpu.VMEM_SHARED((n_pad, DEG_L), jnp.float32),
            pltpu.SemaphoreType.DMA,
            pltpu.SemaphoreType.DMA,
        ],
        compiler_params=_SC_PARAMS,
    )
    def deg_kernel(dst_hbm, out_hbm, idx_v, ones_v, zb_v, acc_sh, sem0, sem1):
        cid = lax.axis_index("c")
        sid = lax.axis_index("s")
        wid = sid * NC + cid
        stripe = pl.ds(sid * rpt, rpt)
        icopy = pltpu.async_copy(dst_hbm.at[wid], idx_v, sem1)
        _fill(ones_v, BLK, DEG_L, 1.0)
        _fill(zb_v, CH, DEG_L, 0.0)
        _zero_stripe(zb_v, acc_sh, sid * rpt, rpt, sem0)
        icopy.wait()
        plsc.subcore_barrier()

        k_ahead = 8

        @pl.loop(0, r)
        def _(j):
            pltpu.async_copy(ones_v, acc_sh.at[idx_v.at[j]], sem1, add=True)

            @pl.when(j >= k_ahead)
            def _():
                pltpu.make_async_copy(ones_v, acc_sh.at[idx_v.at[j]], sem1).wait()

        @pl.loop(0, k_ahead)
        def _(j):
            pltpu.make_async_copy(ones_v, acc_sh.at[idx_v.at[j]], sem1).wait()

        plsc.subcore_barrier()
        pltpu.async_copy(acc_sh.at[stripe], out_hbm.at[cid, stripe], sem0).wait()

    return deg_kernel


NBUF = 4
LOOK = 2


def _agg_kernel(n_pad, r, dhh):
    rpt = n_pad // NS
    assert r % NBUF == 0 and r > NBUF and rpt % CH == 0

    @functools.partial(
        pl.kernel,
        out_type=[
            jax.ShapeDtypeStruct((NC, n_pad, dhh), jnp.float32),
            jax.ShapeDtypeStruct((NC, n_pad, dhh), jnp.float32),
            jax.ShapeDtypeStruct((n_pad, dhh), jnp.float32),
        ],
        mesh=_sc_mesh(),
        scratch_types=[
            pltpu.VMEM((r, BLK), jnp.int32),
            pltpu.VMEM((r, BLK), jnp.int32),
            pltpu.VMEM((NBUF, BLK, dhh), jnp.float32),
            pltpu.VMEM((CH, dhh), jnp.float32),
            pltpu.VMEM((CH, 2 * dhh), jnp.float32),
            pltpu.VMEM((CH, DEG_L), jnp.float32),
            pltpu.VMEM((CH, DEG_L), jnp.float32),
            pltpu.VMEM((CH, dhh), jnp.float32),
            pltpu.VMEM((rpt, DEG_L), jnp.float32),
            pltpu.VMEM_SHARED((n_pad, dhh), jnp.float32),
            pltpu.VMEM_SHARED((n_pad, dhh), jnp.float32),
            pltpu.SemaphoreType.DMA,
            pltpu.SemaphoreType.DMA,
            pltpu.SemaphoreType.DMA,
            pltpu.SemaphoreType.DMA((NBUF,)),
            pltpu.SemaphoreType.DMA((NBUF,)),
        ],
        compiler_params=_SC_PARAMS,
    )
    def agg_kernel(h_hbm, degp_hbm, src_hbm, dst_hbm,
                   out0_hbm, out1_hbm, dinv_hbm,
                   src_v, dst_v, rows_v, zb_v, hbuf, dp0, dp1, gbuf, dinv_v,
                   acc_sh, g_sh, sem0, sem1, sem2, gsem, ssem):
        cid = lax.axis_index("c")
        sid = lax.axis_index("s")
        wid = sid * NC + cid
        s0 = sid * rpt
        stripe = pl.ds(s0, rpt)
        icopy = pltpu.async_copy(src_hbm.at[wid], src_v, sem1)
        jcopy = pltpu.async_copy(dst_hbm.at[wid], dst_v, sem1)
        _fill(zb_v, CH, dhh, 0.0)
        _zero_stripe(zb_v, acc_sh, s0, rpt, sem0)

        def scale_chunks(ph):
            @pl.loop(0, rpt, step=CH)
            def _(c):
                rows = pl.ds(s0 + c, CH)
                hcopy = pltpu.async_copy(h_hbm.at[rows], hbuf, sem2)
                if ph == 0:
                    pltpu.async_copy(degp_hbm.at[0, rows], dp0, sem2).wait()
                    pltpu.async_copy(degp_hbm.at[1, rows], dp1, sem2).wait()
                hcopy.wait()

                @pl.loop(0, CH)
                def _(i):
                    if ph == 0:
                        d = dp0[i, :] + dp1[i, :] + jnp.full(
                            (VL,), 1.0, jnp.float32)
                        y = _rsqrt16(d)
                        dinv_v[c + i, :] = y
                    else:
                        y = dinv_v[c + i, :]
                    off = ph * 2 * VL
                    gbuf[i, pl.ds(0, VL)] = y * hbuf[i, pl.ds(off, VL)]
                    gbuf[i, pl.ds(VL, VL)] = y * hbuf[i, pl.ds(off + VL, VL)]

                pltpu.sync_copy(gbuf, g_sh.at[rows])

        scale_chunks(0)

        @pl.loop(0, rpt, step=CH)
        def _(c):
            @pl.loop(0, CH)
            def _(i):
                y = dinv_v[c + i, :]
                gbuf[i, pl.ds(0, VL)] = y
                gbuf[i, pl.ds(VL, VL)] = y

            pltpu.sync_copy(gbuf, dinv_hbm.at[pl.ds(s0 + c, CH)])
        icopy.wait()
        jcopy.wait()

        for ph in range(2):
            out_hbm = out0_hbm if ph == 0 else out1_hbm
            plsc.subcore_barrier()

            for b in range(LOOK):
                pltpu.async_copy(g_sh.at[src_v.at[b]], rows_v.at[b],
                                 gsem.at[b])

            @pl.loop(0, r, step=NBUF)
            def _(i):
                for b in range(NBUF):
                    j = i + b
                    pn = (b + LOOK) % NBUF

                    @pl.when(jnp.logical_and(j + LOOK < r, j >= LOOK))
                    def _():
                        pltpu.make_async_copy(
                            rows_v.at[pn], acc_sh.at[dst_v.at[j - LOOK]],
                            ssem.at[pn],
                        ).wait()

                    @pl.when(j + LOOK < r)
                    def _():
                        pltpu.async_copy(
                            g_sh.at[src_v.at[j + LOOK]], rows_v.at[pn],
                            gsem.at[pn],
                        )

                    pltpu.make_async_copy(
                        g_sh.at[src_v.at[j]], rows_v.at[b], gsem.at[b]
                    ).wait()
                    pltpu.async_copy(
                        rows_v.at[b], acc_sh.at[dst_v.at[j]], ssem.at[b],
                        add=True,
                    )

            for b in range(NBUF):
                pltpu.make_async_copy(
                    rows_v.at[b], acc_sh.at[dst_v.at[b]], ssem.at[b]
                ).wait()

            plsc.subcore_barrier()
            if ph == 0:
                pltpu.async_copy(acc_sh.at[stripe], out_hbm.at[cid, stripe],
                                 sem1).wait()
                scale_chunks(1)
                _zero_stripe(zb_v, acc_sh, s0, rpt, sem0)
            else:
                pltpu.async_copy(acc_sh.at[stripe], out_hbm.at[cid, stripe],
                                 sem1).wait()

    return agg_kernel


def _matmul(x, w, n, n_pad, d_in, dh):
    def body(x_ref, w_ref, h_ref):
        h_ref[...] = jnp.dot(x_ref[...], w_ref[...],
                             preferred_element_type=jnp.float32)

    return pl.pallas_call(
        body,
        grid=(n // BM,),
        in_specs=[
            pl.BlockSpec((BM, d_in), lambda i: (i, 0)),
            pl.BlockSpec((d_in, dh), lambda i: (0, 0)),
        ],
        out_specs=pl.BlockSpec((BM, dh), lambda i: (i, 0)),
        out_shape=jax.ShapeDtypeStruct((n_pad, dh), jnp.float32),
    )(x, w)


def _combine(accp0, accp1, h, dinv, b2, n, dh):
    dhh = dh // 2

    def body(a0_ref, a1_ref, h_ref, dinv_ref, b_ref, o_ref):
        d32 = dinv_ref[...]
        h = h_ref[...]
        o_ref[:, :dhh] = (d32 * (a0_ref[0] + a0_ref[1] + d32 * h[:, :dhh])
                          + b_ref[:, :dhh])
        o_ref[:, dhh:] = (d32 * (a1_ref[0] + a1_ref[1] + d32 * h[:, dhh:])
                          + b_ref[:, dhh:])

    return pl.pallas_call(
        body,
        grid=(n // BM,),
        in_specs=[
            pl.BlockSpec((NC, BM, dhh), lambda i: (0, i, 0)),
            pl.BlockSpec((NC, BM, dhh), lambda i: (0, i, 0)),
            pl.BlockSpec((BM, dh), lambda i: (i, 0)),
            pl.BlockSpec((BM, dhh), lambda i: (i, 0)),
            pl.BlockSpec((1, dh), lambda i: (0, 0)),
        ],
        out_specs=pl.BlockSpec((BM, dh), lambda i: (i, 0)),
        out_shape=jax.ShapeDtypeStruct((n, dh), jnp.float32),
    )(accp0, accp1, h, dinv, b2)


def kernel(x, edge_index, W, b):
    n, d_in = x.shape
    dh = W.shape[1]
    e = edge_index.shape[1]

    n_pad = -(-(n + 1) // (NS * CH)) * (NS * CH)

    per_w = -(-e // NW)
    r = -(-per_w // BLK)
    r = ((r + NBUF - 1) // NBUF) * NBUF
    e_pad = NW * r * BLK

    ei = edge_index.astype(jnp.int32)
    sink = n + jnp.arange(e_pad - e, dtype=jnp.int32) % (n_pad - n)
    ei = jnp.concatenate([ei, jnp.stack([sink, sink])], axis=1)
    src3 = ei[0].reshape(NW, r, BLK)
    dst3 = ei[1].reshape(NW, r, BLK)

    b2 = b.reshape(1, dh).astype(jnp.float32)

    degp = _deg_kernel(n_pad, r)(dst3)
    h = _matmul(x, W, n, n_pad, d_in, dh)
    accp0, accp1, dinv = _agg_kernel(n_pad, r, dh // 2)(h, degp, src3, dst3)
    return _combine(accp0, accp1, h, dinv, b2, n, dh)


# speedup vs baseline: 1.1397x; 1.1397x over previous
import functools

import jax
import jax.numpy as jnp
from jax import lax
from jax.experimental import pallas as pl
from jax.experimental.pallas import tpu as pltpu
from jax.experimental.pallas import tpu_sc as plsc

NC = 2
NS = 16
NW = NC * NS
BLK = 256
DEG_L = 16
BM = 2000
VL = 16


def _sc_mesh():
    return plsc.VectorSubcoreMesh(core_axis_name="c", subcore_axis_name="s")


_SC_PARAMS = pltpu.CompilerParams(use_tc_tiling_on_sc=False)


def _fill(ref, rows, cols, value):
    vec = jnp.full((VL,), value, jnp.float32)

    @pl.loop(0, rows)
    def _(i):
        for l in range(cols // VL):
            ref[i, pl.ds(l * VL, VL)] = vec


def _zero_stripe(zbuf, acc_sh, base, rows, sem):
    @pl.loop(0, rows, step=BLK)
    def _(i):
        pltpu.async_copy(zbuf, acc_sh.at[pl.ds(base + i, BLK)], sem)

    @pl.loop(0, rows, step=BLK)
    def _(i):
        pltpu.make_async_copy(zbuf, acc_sh.at[pl.ds(base + i, BLK)], sem).wait()


def _deg_kernel(n_pad, r):
    rpt = n_pad // NS

    @functools.partial(
        pl.kernel,
        out_type=jax.ShapeDtypeStruct((NC, n_pad, DEG_L), jnp.float32),
        mesh=_sc_mesh(),
        scratch_types=[
            pltpu.VMEM((r, BLK), jnp.int32),
            pltpu.VMEM((BLK, DEG_L), jnp.float32),
            pltpu.VMEM((BLK, DEG_L), jnp.float32),
            pltpu.VMEM_SHARED((n_pad, DEG_L), jnp.float32),
            pltpu.SemaphoreType.DMA,
            pltpu.SemaphoreType.DMA,
        ],
        compiler_params=_SC_PARAMS,
    )
    def deg_kernel(dst_hbm, out_hbm, idx_v, ones_v, zb_v, acc_sh, sem0, sem1):
        cid = lax.axis_index("c")
        sid = lax.axis_index("s")
        wid = sid * NC + cid
        stripe = pl.ds(sid * rpt, rpt)
        icopy = pltpu.async_copy(dst_hbm.at[wid], idx_v, sem1)
        _fill(ones_v, BLK, DEG_L, 1.0)
        _fill(zb_v, BLK, DEG_L, 0.0)
        _zero_stripe(zb_v, acc_sh, sid * rpt, rpt, sem0)
        icopy.wait()
        plsc.subcore_barrier()

        k_ahead = 8

        @pl.loop(0, r)
        def _(j):
            pltpu.async_copy(ones_v, acc_sh.at[idx_v.at[j]], sem1, add=True)

            @pl.when(j >= k_ahead)
            def _():
                pltpu.make_async_copy(ones_v, acc_sh.at[idx_v.at[j]], sem1).wait()

        @pl.loop(0, k_ahead)
        def _(j):
            pltpu.make_async_copy(ones_v, acc_sh.at[idx_v.at[j]], sem1).wait()

        plsc.subcore_barrier()
        pltpu.async_copy(acc_sh.at[stripe], out_hbm.at[cid, stripe], sem0).wait()

    return deg_kernel


NBUF = 4
LOOK = 2


def _agg_kernel(n_pad, r, dhh):
    rpt = n_pad // NS
    assert r % NBUF == 0 and r > NBUF

    @functools.partial(
        pl.kernel,
        out_type=[
            jax.ShapeDtypeStruct((NC, n_pad, dhh), jnp.float32),
            jax.ShapeDtypeStruct((NC, n_pad, dhh), jnp.float32),
        ],
        mesh=_sc_mesh(),
        scratch_types=[
            pltpu.VMEM((r, BLK), jnp.int32),
            pltpu.VMEM((r, BLK), jnp.int32),
            pltpu.VMEM((NBUF, BLK, dhh), jnp.float32),
            pltpu.VMEM((BLK, dhh), jnp.float32),
            pltpu.VMEM_SHARED((n_pad, dhh), jnp.float32),
            pltpu.VMEM_SHARED((n_pad, dhh), jnp.float32),
            pltpu.SemaphoreType.DMA,
            pltpu.SemaphoreType.DMA,
            pltpu.SemaphoreType.DMA((NBUF,)),
            pltpu.SemaphoreType.DMA((NBUF,)),
        ],
        compiler_params=_SC_PARAMS,
    )
    def agg_kernel(g0_hbm, g1_hbm, src_hbm, dst_hbm, out0_hbm, out1_hbm,
                   src_v, dst_v, rows_v, zb_v, acc_sh, g_sh,
                   sem0, sem1, gsem, ssem):
        cid = lax.axis_index("c")
        sid = lax.axis_index("s")
        wid = sid * NC + cid
        stripe = pl.ds(sid * rpt, rpt)
        icopy = pltpu.async_copy(src_hbm.at[wid], src_v, sem1)
        jcopy = pltpu.async_copy(dst_hbm.at[wid], dst_v, sem1)
        _fill(zb_v, BLK, dhh, 0.0)
        gcopy = pltpu.async_copy(g0_hbm.at[stripe], g_sh.at[stripe], sem0)
        _zero_stripe(zb_v, acc_sh, sid * rpt, rpt, sem0)
        icopy.wait()
        jcopy.wait()
        gcopy.wait()

        for ph in range(2):
            g_next = g1_hbm
            out_hbm = out0_hbm if ph == 0 else out1_hbm
            plsc.subcore_barrier()

            for b in range(LOOK):
                pltpu.async_copy(g_sh.at[src_v.at[b]], rows_v.at[b],
                                 gsem.at[b])

            @pl.loop(0, r, step=NBUF)
            def _(i):
                for b in range(NBUF):
                    j = i + b
                    pn = (b + LOOK) % NBUF

                    @pl.when(jnp.logical_and(j + LOOK < r, j >= LOOK))
                    def _():
                        pltpu.make_async_copy(
                            rows_v.at[pn], acc_sh.at[dst_v.at[j - LOOK]],
                            ssem.at[pn],
                        ).wait()

                    @pl.when(j + LOOK < r)
                    def _():
                        pltpu.async_copy(
                            g_sh.at[src_v.at[j + LOOK]], rows_v.at[pn],
                            gsem.at[pn],
                        )

                    pltpu.make_async_copy(
                        g_sh.at[src_v.at[j]], rows_v.at[b], gsem.at[b]
                    ).wait()
                    pltpu.async_copy(
                        rows_v.at[b], acc_sh.at[dst_v.at[j]], ssem.at[b],
                        add=True,
                    )

            for b in range(NBUF):
                pltpu.make_async_copy(
                    rows_v.at[b], acc_sh.at[dst_v.at[b]], ssem.at[b]
                ).wait()

            plsc.subcore_barrier()
            if ph == 0:
                gcopy = pltpu.async_copy(g_next.at[stripe], g_sh.at[stripe],
                                         sem0)
                pltpu.async_copy(acc_sh.at[stripe], out_hbm.at[cid, stripe],
                                 sem1).wait()
                _zero_stripe(zb_v, acc_sh, sid * rpt, rpt, sem0)
                gcopy.wait()
            else:
                pltpu.async_copy(acc_sh.at[stripe], out_hbm.at[cid, stripe],
                                 sem1).wait()

    return agg_kernel


def _matmul(x, w, n, d_in, dh):
    def body(x_ref, w_ref, h_ref):
        h_ref[...] = jnp.dot(x_ref[...], w_ref[...],
                             preferred_element_type=jnp.float32)

    return pl.pallas_call(
        body,
        grid=(n // BM,),
        in_specs=[
            pl.BlockSpec((BM, d_in), lambda i: (i, 0)),
            pl.BlockSpec((d_in, dh), lambda i: (0, 0)),
        ],
        out_specs=pl.BlockSpec((BM, dh), lambda i: (i, 0)),
        out_shape=jax.ShapeDtypeStruct((n, dh), jnp.float32),
    )(x, w)


def _scale(h, degp, n, n_pad, dh):
    dhh = dh // 2

    def body(h_ref, degp_ref, g0_ref, g1_ref, dinv_ref):
        h = h_ref[...]
        deg = jnp.sum(degp_ref[...], axis=(0, 2)) * (1.0 / DEG_L) + 1.0
        d32 = jnp.broadcast_to(lax.rsqrt(deg)[:, None], (BM, dhh))
        g0_ref[...] = d32 * h[:, :dhh]
        g1_ref[...] = d32 * h[:, dhh:]
        dinv_ref[...] = d32

    return pl.pallas_call(
        body,
        grid=(n // BM,),
        in_specs=[
            pl.BlockSpec((BM, dh), lambda i: (i, 0)),
            pl.BlockSpec((NC, BM, DEG_L), lambda i: (0, i, 0)),
        ],
        out_specs=[
            pl.BlockSpec((BM, dhh), lambda i: (i, 0)),
            pl.BlockSpec((BM, dhh), lambda i: (i, 0)),
            pl.BlockSpec((BM, dhh), lambda i: (i, 0)),
        ],
        out_shape=[
            jax.ShapeDtypeStruct((n_pad, dhh), jnp.float32),
            jax.ShapeDtypeStruct((n_pad, dhh), jnp.float32),
            jax.ShapeDtypeStruct((n, dhh), jnp.float32),
        ],
    )(h, degp)


def _combine(accp0, accp1, g0, g1, dinv, b2, n, dh):
    dhh = dh // 2

    def body(a0_ref, a1_ref, g0_ref, g1_ref, dinv_ref, b_ref, o_ref):
        d32 = dinv_ref[...]
        o_ref[:, :dhh] = d32 * (a0_ref[0] + a0_ref[1] + g0_ref[...]) + b_ref[:, :dhh]
        o_ref[:, dhh:] = d32 * (a1_ref[0] + a1_ref[1] + g1_ref[...]) + b_ref[:, dhh:]

    return pl.pallas_call(
        body,
        grid=(n // BM,),
        in_specs=[
            pl.BlockSpec((NC, BM, dhh), lambda i: (0, i, 0)),
            pl.BlockSpec((NC, BM, dhh), lambda i: (0, i, 0)),
            pl.BlockSpec((BM, dhh), lambda i: (i, 0)),
            pl.BlockSpec((BM, dhh), lambda i: (i, 0)),
            pl.BlockSpec((BM, dhh), lambda i: (i, 0)),
            pl.BlockSpec((1, dh), lambda i: (0, 0)),
        ],
        out_specs=pl.BlockSpec((BM, dh), lambda i: (i, 0)),
        out_shape=jax.ShapeDtypeStruct((n, dh), jnp.float32),
    )(accp0, accp1, g0, g1, dinv, b2)


def kernel(x, edge_index, W, b):
    n, d_in = x.shape
    dh = W.shape[1]
    e = edge_index.shape[1]

    n_pad = -(-(n + 1) // (NS * BLK)) * (NS * BLK)

    per_w = -(-e // NW)
    r = -(-per_w // BLK)
    r = ((r + NBUF - 1) // NBUF) * NBUF
    e_pad = NW * r * BLK

    ei = edge_index.astype(jnp.int32)
    sink = n + jnp.arange(e_pad - e, dtype=jnp.int32) % (n_pad - n)
    ei = jnp.concatenate([ei, jnp.stack([sink, sink])], axis=1)
    src3 = ei[0].reshape(NW, r, BLK)
    dst3 = ei[1].reshape(NW, r, BLK)

    b2 = b.reshape(1, dh).astype(jnp.float32)

    degp = _deg_kernel(n_pad, r)(dst3)
    h = _matmul(x, W, n, d_in, dh)
    g0, g1, dinv = _scale(h, degp, n, n_pad, dh)
    accp0, accp1 = _agg_kernel(n_pad, r, dh // 2)(g0, g1, src3, dst3)
    return _combine(accp0, accp1, g0, g1, dinv, b2, n, dh)


# speedup vs baseline: 1.1520x; 1.0108x over previous
import functools

import jax
import jax.numpy as jnp
from jax import lax
from jax.experimental import pallas as pl
from jax.experimental.pallas import tpu as pltpu
from jax.experimental.pallas import tpu_sc as plsc

NC = 2
NS = 16
NW = NC * NS
BLK = 256
DEG_L = 16
BM = 5000
VL = 16


def _sc_mesh():
    return plsc.VectorSubcoreMesh(core_axis_name="c", subcore_axis_name="s")


_SC_PARAMS = pltpu.CompilerParams(use_tc_tiling_on_sc=False)


def _fill(ref, rows, cols, value):
    vec = jnp.full((VL,), value, jnp.float32)

    @pl.loop(0, rows)
    def _(i):
        for l in range(cols // VL):
            ref[i, pl.ds(l * VL, VL)] = vec


def _zero_stripe(zbuf, acc_sh, base, rows, sem):
    @pl.loop(0, rows, step=BLK)
    def _(i):
        pltpu.async_copy(zbuf, acc_sh.at[pl.ds(base + i, BLK)], sem)

    @pl.loop(0, rows, step=BLK)
    def _(i):
        pltpu.make_async_copy(zbuf, acc_sh.at[pl.ds(base + i, BLK)], sem).wait()


def _deg_kernel(n_pad, r):
    rpt = n_pad // NS

    @functools.partial(
        pl.kernel,
        out_type=jax.ShapeDtypeStruct((NC, n_pad, DEG_L), jnp.float32),
        mesh=_sc_mesh(),
        scratch_types=[
            pltpu.VMEM((r, BLK), jnp.int32),
            pltpu.VMEM((BLK, DEG_L), jnp.float32),
            pltpu.VMEM((BLK, DEG_L), jnp.float32),
            pltpu.VMEM_SHARED((n_pad, DEG_L), jnp.float32),
            pltpu.SemaphoreType.DMA,
            pltpu.SemaphoreType.DMA,
        ],
        compiler_params=_SC_PARAMS,
    )
    def deg_kernel(dst_hbm, out_hbm, idx_v, ones_v, zb_v, acc_sh, sem0, sem1):
        cid = lax.axis_index("c")
        sid = lax.axis_index("s")
        wid = sid * NC + cid
        stripe = pl.ds(sid * rpt, rpt)
        icopy = pltpu.async_copy(dst_hbm.at[wid], idx_v, sem1)
        _fill(ones_v, BLK, DEG_L, 1.0)
        _fill(zb_v, BLK, DEG_L, 0.0)
        _zero_stripe(zb_v, acc_sh, sid * rpt, rpt, sem0)
        icopy.wait()
        plsc.subcore_barrier()

        k_ahead = 8

        @pl.loop(0, r)
        def _(j):
            pltpu.async_copy(ones_v, acc_sh.at[idx_v.at[j]], sem1, add=True)

            @pl.when(j >= k_ahead)
            def _():
                pltpu.make_async_copy(ones_v, acc_sh.at[idx_v.at[j]], sem1).wait()

        @pl.loop(0, k_ahead)
        def _(j):
            pltpu.make_async_copy(ones_v, acc_sh.at[idx_v.at[j]], sem1).wait()

        plsc.subcore_barrier()
        pltpu.async_copy(acc_sh.at[stripe], out_hbm.at[cid, stripe], sem0).wait()

    return deg_kernel


NBUF = 4
LOOK = 2


def _agg_kernel(n_pad, r, dhh):
    rpt = n_pad // NS
    assert r % NBUF == 0 and r > NBUF

    @functools.partial(
        pl.kernel,
        out_type=[
            jax.ShapeDtypeStruct((NC, n_pad, dhh), jnp.float32),
            jax.ShapeDtypeStruct((NC, n_pad, dhh), jnp.float32),
        ],
        mesh=_sc_mesh(),
        scratch_types=[
            pltpu.VMEM((r, BLK), jnp.int32),
            pltpu.VMEM((r, BLK), jnp.int32),
            pltpu.VMEM((NBUF, BLK, dhh), jnp.float32),
            pltpu.VMEM((BLK, dhh), jnp.float32),
            pltpu.VMEM_SHARED((n_pad, dhh), jnp.float32),
            pltpu.VMEM_SHARED((n_pad, dhh), jnp.float32),
            pltpu.SemaphoreType.DMA,
            pltpu.SemaphoreType.DMA,
            pltpu.SemaphoreType.DMA((NBUF,)),
            pltpu.SemaphoreType.DMA((NBUF,)),
        ],
        compiler_params=_SC_PARAMS,
    )
    def agg_kernel(g0_hbm, g1_hbm, src_hbm, dst_hbm, out0_hbm, out1_hbm,
                   src_v, dst_v, rows_v, zb_v, acc_sh, g_sh,
                   sem0, sem1, gsem, ssem):
        cid = lax.axis_index("c")
        sid = lax.axis_index("s")
        wid = sid * NC + cid
        stripe = pl.ds(sid * rpt, rpt)
        icopy = pltpu.async_copy(src_hbm.at[wid], src_v, sem1)
        jcopy = pltpu.async_copy(dst_hbm.at[wid], dst_v, sem1)
        _fill(zb_v, BLK, dhh, 0.0)
        gcopy = pltpu.async_copy(g0_hbm.at[stripe], g_sh.at[stripe], sem0)
        _zero_stripe(zb_v, acc_sh, sid * rpt, rpt, sem0)
        icopy.wait()
        jcopy.wait()
        gcopy.wait()

        for ph in range(2):
            g_next = g1_hbm
            out_hbm = out0_hbm if ph == 0 else out1_hbm
            plsc.subcore_barrier()

            for b in range(LOOK):
                pltpu.async_copy(g_sh.at[src_v.at[b]], rows_v.at[b],
                                 gsem.at[b])

            @pl.loop(0, r, step=NBUF)
            def _(i):
                for b in range(NBUF):
                    j = i + b
                    pn = (b + LOOK) % NBUF

                    @pl.when(jnp.logical_and(j + LOOK < r, j >= LOOK))
                    def _():
                        pltpu.make_async_copy(
                            rows_v.at[pn], acc_sh.at[dst_v.at[j - LOOK]],
                            ssem.at[pn],
                        ).wait()

                    @pl.when(j + LOOK < r)
                    def _():
                        pltpu.async_copy(
                            g_sh.at[src_v.at[j + LOOK]], rows_v.at[pn],
                            gsem.at[pn],
                        )

                    pltpu.make_async_copy(
                        g_sh.at[src_v.at[j]], rows_v.at[b], gsem.at[b]
                    ).wait()
                    pltpu.async_copy(
                        rows_v.at[b], acc_sh.at[dst_v.at[j]], ssem.at[b],
                        add=True,
                    )

            for b in range(NBUF):
                pltpu.make_async_copy(
                    rows_v.at[b], acc_sh.at[dst_v.at[b]], ssem.at[b]
                ).wait()

            plsc.subcore_barrier()
            if ph == 0:
                gcopy = pltpu.async_copy(g_next.at[stripe], g_sh.at[stripe],
                                         sem0)
                pltpu.async_copy(acc_sh.at[stripe], out_hbm.at[cid, stripe],
                                 sem1).wait()
                _zero_stripe(zb_v, acc_sh, sid * rpt, rpt, sem0)
                gcopy.wait()
            else:
                pltpu.async_copy(acc_sh.at[stripe], out_hbm.at[cid, stripe],
                                 sem1).wait()

    return agg_kernel


def _matmul(x, w, n, d_in, dh):
    def body(x_ref, w_ref, h_ref):
        h_ref[...] = jnp.dot(x_ref[...], w_ref[...],
                             preferred_element_type=jnp.float32)

    return pl.pallas_call(
        body,
        grid=(n // BM,),
        in_specs=[
            pl.BlockSpec((BM, d_in), lambda i: (i, 0)),
            pl.BlockSpec((d_in, dh), lambda i: (0, 0)),
        ],
        out_specs=pl.BlockSpec((BM, dh), lambda i: (i, 0)),
        out_shape=jax.ShapeDtypeStruct((n, dh), jnp.float32),
    )(x, w)


def _scale(h, degp, n, n_pad, dh):
    dhh = dh // 2

    def body(h_ref, degp_ref, g0_ref, g1_ref, dinv_ref):
        h = h_ref[...]
        deg = jnp.sum(degp_ref[...], axis=(0, 2)) * (1.0 / DEG_L) + 1.0
        d32 = jnp.broadcast_to(lax.rsqrt(deg)[:, None], (BM, dhh))
        g0_ref[...] = d32 * h[:, :dhh]
        g1_ref[...] = d32 * h[:, dhh:]
        dinv_ref[...] = d32

    return pl.pallas_call(
        body,
        grid=(n // BM,),
        in_specs=[
            pl.BlockSpec((BM, dh), lambda i: (i, 0)),
            pl.BlockSpec((NC, BM, DEG_L), lambda i: (0, i, 0)),
        ],
        out_specs=[
            pl.BlockSpec((BM, dhh), lambda i: (i, 0)),
            pl.BlockSpec((BM, dhh), lambda i: (i, 0)),
            pl.BlockSpec((BM, dhh), lambda i: (i, 0)),
        ],
        out_shape=[
            jax.ShapeDtypeStruct((n_pad, dhh), jnp.float32),
            jax.ShapeDtypeStruct((n_pad, dhh), jnp.float32),
            jax.ShapeDtypeStruct((n, dhh), jnp.float32),
        ],
    )(h, degp)


def _combine(accp0, accp1, g0, g1, dinv, b2, n, dh):
    dhh = dh // 2

    def body(a0_ref, a1_ref, g0_ref, g1_ref, dinv_ref, b_ref, o_ref):
        d32 = dinv_ref[...]
        o_ref[:, :dhh] = d32 * (a0_ref[0] + a0_ref[1] + g0_ref[...]) + b_ref[:, :dhh]
        o_ref[:, dhh:] = d32 * (a1_ref[0] + a1_ref[1] + g1_ref[...]) + b_ref[:, dhh:]

    return pl.pallas_call(
        body,
        grid=(n // BM,),
        in_specs=[
            pl.BlockSpec((NC, BM, dhh), lambda i: (0, i, 0)),
            pl.BlockSpec((NC, BM, dhh), lambda i: (0, i, 0)),
            pl.BlockSpec((BM, dhh), lambda i: (i, 0)),
            pl.BlockSpec((BM, dhh), lambda i: (i, 0)),
            pl.BlockSpec((BM, dhh), lambda i: (i, 0)),
            pl.BlockSpec((1, dh), lambda i: (0, 0)),
        ],
        out_specs=pl.BlockSpec((BM, dh), lambda i: (i, 0)),
        out_shape=jax.ShapeDtypeStruct((n, dh), jnp.float32),
    )(accp0, accp1, g0, g1, dinv, b2)


def kernel(x, edge_index, W, b):
    n, d_in = x.shape
    dh = W.shape[1]
    e = edge_index.shape[1]

    n_pad = -(-(n + 1) // (NS * BLK)) * (NS * BLK)

    per_w = -(-e // NW)
    r = -(-per_w // BLK)
    r = ((r + NBUF - 1) // NBUF) * NBUF
    e_pad = NW * r * BLK

    ei = edge_index.astype(jnp.int32)
    sink = n + jnp.arange(e_pad - e, dtype=jnp.int32) % (n_pad - n)
    ei = jnp.concatenate([ei, jnp.stack([sink, sink])], axis=1)
    src3 = ei[0].reshape(NW, r, BLK)
    dst3 = ei[1].reshape(NW, r, BLK)

    b2 = b.reshape(1, dh).astype(jnp.float32)

    degp = _deg_kernel(n_pad, r)(dst3)
    h = _matmul(x, W, n, d_in, dh)
    g0, g1, dinv = _scale(h, degp, n, n_pad, dh)
    accp0, accp1 = _agg_kernel(n_pad, r, dh // 2)(g0, g1, src3, dst3)
    return _combine(accp0, accp1, g0, g1, dinv, b2, n, dh)
